# baseline (speedup 1.0000x reference)
import functools

import jax
import jax.numpy as jnp
from jax import lax
from jax.experimental import pallas as pl
from jax.experimental.pallas import tpu as pltpu
from jax.experimental.pallas import tpu_sc as plsc

N = 10000
E = 320000
C_IN = 128
C_OUT = 128
KV = 27

NC, NS = 2, 16
NW = NC * NS
BE = 128
CH0, CH1 = 160, 0
TCH = NS * (CH0 + CH1)
PH = 40
EPAD = TCH * BE
NPAD = N + 112
RPT = NPAD // NS


def _mm_body(x_ref, w_ref, o_ref):
    for k in range(KV):
        o_ref[k] = jnp.dot(x_ref[...], w_ref[k],
                           preferred_element_type=jnp.float32)


def _bn_body(p_ref, g_ref, b_ref, o_ref):
    s = p_ref[:N, :]
    mean = jnp.mean(s, axis=0, keepdims=True)
    var = jnp.mean(s * s, axis=0, keepdims=True) - mean * mean
    inv = lax.rsqrt(var + 1e-5)
    o_ref[...] = jnp.maximum((s - mean) * inv * g_ref[...] + b_ref[...], 0.0)


def _sc_scatter_body(xw_hbm, g_hbm, d_hbm, z_hbm, out_hbm,
                     g_v, d_v, rows0_v, rows1_v, acc_sh, sem0, sem1):
    c = lax.axis_index("c")
    s = lax.axis_index("s")

    rows = (rows0_v, rows1_v)
    sems = (sem0, sem1)

    def _phase(row0):
        pltpu.sync_copy(g_hbm.at[pl.ds(row0, PH)], g_v)
        pltpu.sync_copy(d_hbm.at[pl.ds(row0, PH)], d_v)

        pltpu.async_copy(xw_hbm.at[g_v.at[0]], rows[0], sems[0])

        def body(j, carry):
            for b in range(2):
                i = 2 * j + b
                pltpu.make_async_copy(xw_hbm.at[g_v.at[i]],
                                      rows[b], sems[b]).wait()

                @pl.when(i + 1 < PH)
                def _():
                    pltpu.async_copy(xw_hbm.at[g_v.at[i + 1]],
                                     rows[1 - b], sems[1 - b])

                pltpu.sync_copy(rows[b], acc_sh.at[d_v.at[i]], add=True)
            return carry

        lax.fori_loop(0, PH // 2, body, 0)

    @pl.when(c == 0)
    def _():
        pltpu.sync_copy(z_hbm.at[pl.ds(s * RPT, RPT)],
                        acc_sh.at[pl.ds(s * RPT, RPT)])
        plsc.subcore_barrier()

        for h in range(CH0 // PH):
            _phase(s * CH0 + h * PH)

        plsc.subcore_barrier()
        pltpu.sync_copy(acc_sh.at[pl.ds(s * RPT, RPT)],
                        out_hbm.at[pl.ds(s * RPT, RPT)])


def kernel(x, edge_index, kernel_id, W, gamma, beta):
    src = edge_index[0].astype(jnp.int32)
    dst = edge_index[1].astype(jnp.int32)
    kid = kernel_id.astype(jnp.int32)

    n_blk = 10
    bn = N // n_blk
    xw = pl.pallas_call(
        _mm_body,
        grid=(n_blk,),
        in_specs=[pl.BlockSpec((bn, C_IN), lambda i: (i, 0)),
                  pl.BlockSpec((KV, C_IN, C_OUT), lambda i: (0, 0, 0))],
        out_specs=pl.BlockSpec((KV, bn, C_OUT), lambda i: (0, i, 0)),
        out_shape=jax.ShapeDtypeStruct((KV, N, C_OUT), jnp.float32),
    )(x, W)
    xw_rows = xw.reshape(KV * N, C_OUT)

    g = kid * N + src
    pad = EPAD - E
    g2 = jnp.concatenate([g, jnp.zeros((pad,), jnp.int32)]).reshape(TCH, BE)
    d2 = jnp.concatenate([dst, jnp.full((pad,), N, jnp.int32)]).reshape(TCH, BE)
    zeros = jnp.zeros((NPAD, C_OUT), jnp.float32)

    mesh = plsc.VectorSubcoreMesh(core_axis_name="c", subcore_axis_name="s")
    partials = pl.kernel(
        _sc_scatter_body,
        out_type=jax.ShapeDtypeStruct((NPAD, C_OUT), jnp.float32),
        mesh=mesh,
        scratch_types=[
            pltpu.VMEM((PH, BE), jnp.int32),
            pltpu.VMEM((PH, BE), jnp.int32),
            pltpu.VMEM((BE, C_OUT), jnp.float32),
            pltpu.VMEM((BE, C_OUT), jnp.float32),
            pltpu.VMEM_SHARED((NPAD, C_OUT), jnp.float32),
            pltpu.SemaphoreType.DMA,
            pltpu.SemaphoreType.DMA,
        ],
    )(xw_rows, g2, d2, zeros)

    out = pl.pallas_call(
        _bn_body,
        in_specs=[pl.BlockSpec((NPAD, C_OUT), lambda: (0, 0)),
                  pl.BlockSpec((1, C_OUT), lambda: (0, 0)),
                  pl.BlockSpec((1, C_OUT), lambda: (0, 0))],
        out_specs=pl.BlockSpec((N, C_OUT), lambda: (0, 0)),
        out_shape=jax.ShapeDtypeStruct((N, C_OUT), jnp.float32),
    )(partials, gamma.reshape(1, C_OUT), beta.reshape(1, C_OUT))
    return out


# speedup vs baseline: 1.3589x; 1.3589x over previous
import functools

import jax
import jax.numpy as jnp
from jax import lax
from jax.experimental import pallas as pl
from jax.experimental.pallas import tpu as pltpu
from jax.experimental.pallas import tpu_sc as plsc

N = 10000
E = 320000
C_IN = 128
C_OUT = 128
KV = 27

NC, NS = 2, 16
NW = NC * NS
BE = 128
CH0, CH1 = 120, 40
TCH = NS * (CH0 + CH1)
PH = 40
EPAD = TCH * BE
NPAD = N + 112
RPT = NPAD // NS


def _mm_body(x_ref, w_ref, o_ref):
    for k in range(KV):
        o_ref[k] = jnp.dot(x_ref[...], w_ref[k],
                           preferred_element_type=jnp.float32)


def _bn_body(p_ref, g_ref, b_ref, o_ref):
    s = p_ref[0, :N, :] + p_ref[1, :N, :]
    mean = jnp.mean(s, axis=0, keepdims=True)
    var = jnp.mean(s * s, axis=0, keepdims=True) - mean * mean
    inv = lax.rsqrt(var + 1e-5)
    o_ref[...] = jnp.maximum((s - mean) * inv * g_ref[...] + b_ref[...], 0.0)


def _sc_scatter_body(xw_hbm, g_hbm, d_hbm, out_hbm,
                     g_v, d_v, rows0_v, rows1_v, acc_sh, sem0, sem1):
    c = lax.axis_index("c")
    s = lax.axis_index("s")

    rows = (rows0_v, rows1_v)
    sems = (sem0, sem1)

    def zbody(i, carry):
        rows0_v[i >> 3, pl.ds((i & 7) * 16, 16)] = jnp.zeros((16,),
                                                             jnp.float32)
        return carry

    lax.fori_loop(0, BE * 8, zbody, 0)
    for t in range(4):
        pltpu.sync_copy(rows0_v,
                        acc_sh.at[pl.ds(s * RPT + t * BE, BE)])
    pltpu.sync_copy(rows0_v.at[pl.ds(0, RPT - 4 * BE)],
                    acc_sh.at[pl.ds(s * RPT + 4 * BE, RPT - 4 * BE)])
    plsc.subcore_barrier()

    def _phase(row0):
        pltpu.sync_copy(g_hbm.at[pl.ds(row0, PH)], g_v)
        pltpu.sync_copy(d_hbm.at[pl.ds(row0, PH)], d_v)

        pltpu.async_copy(xw_hbm.at[g_v.at[0]], rows[0], sems[0])

        def body(j, carry):
            for b in range(2):
                i = 2 * j + b
                pltpu.make_async_copy(xw_hbm.at[g_v.at[i]],
                                      rows[b], sems[b]).wait()

                @pl.when(i + 1 < PH)
                def _():
                    pltpu.async_copy(xw_hbm.at[g_v.at[i + 1]],
                                     rows[1 - b], sems[1 - b])

                pltpu.sync_copy(rows[b], acc_sh.at[d_v.at[i]], add=True)
            return carry

        lax.fori_loop(0, PH // 2, body, 0)

    @pl.when(c == 0)
    def _():
        for h in range(CH0 // PH):
            _phase(s * CH0 + h * PH)

    @pl.when(c == 1)
    def _():
        for h in range(CH1 // PH):
            _phase(NS * CH0 + s * CH1 + h * PH)

    plsc.subcore_barrier()
    pltpu.sync_copy(acc_sh.at[pl.ds(s * RPT, RPT)],
                    out_hbm.at[c, pl.ds(s * RPT, RPT)])


def kernel(x, edge_index, kernel_id, W, gamma, beta):
    src = edge_index[0].astype(jnp.int32)
    dst = edge_index[1].astype(jnp.int32)
    kid = kernel_id.astype(jnp.int32)

    n_blk = 10
    bn = N // n_blk
    xw = pl.pallas_call(
        _mm_body,
        grid=(n_blk,),
        in_specs=[pl.BlockSpec((bn, C_IN), lambda i: (i, 0)),
                  pl.BlockSpec((KV, C_IN, C_OUT), lambda i: (0, 0, 0))],
        out_specs=pl.BlockSpec((KV, bn, C_OUT), lambda i: (0, i, 0)),
        out_shape=jax.ShapeDtypeStruct((KV, N, C_OUT), jnp.float32),
    )(x, W)
    xw_rows = xw.reshape(KV * N, C_OUT)

    g = kid * N + src
    pad = EPAD - E
    g2 = jnp.concatenate([g, jnp.zeros((pad,), jnp.int32)]).reshape(TCH, BE)
    d2 = jnp.concatenate([dst, jnp.full((pad,), N, jnp.int32)]).reshape(TCH, BE)

    mesh = plsc.VectorSubcoreMesh(core_axis_name="c", subcore_axis_name="s")
    partials = pl.kernel(
        _sc_scatter_body,
        out_type=jax.ShapeDtypeStruct((NC, NPAD, C_OUT), jnp.float32),
        mesh=mesh,
        scratch_types=[
            pltpu.VMEM((PH, BE), jnp.int32),
            pltpu.VMEM((PH, BE), jnp.int32),
            pltpu.VMEM((BE, C_OUT), jnp.float32),
            pltpu.VMEM((BE, C_OUT), jnp.float32),
            pltpu.VMEM_SHARED((NPAD, C_OUT), jnp.float32),
            pltpu.SemaphoreType.DMA,
            pltpu.SemaphoreType.DMA,
        ],
    )(xw_rows, g2, d2)

    out = pl.pallas_call(
        _bn_body,
        in_specs=[pl.BlockSpec((NC, NPAD, C_OUT), lambda: (0, 0, 0)),
                  pl.BlockSpec((1, C_OUT), lambda: (0, 0)),
                  pl.BlockSpec((1, C_OUT), lambda: (0, 0))],
        out_specs=pl.BlockSpec((N, C_OUT), lambda: (0, 0)),
        out_shape=jax.ShapeDtypeStruct((N, C_OUT), jnp.float32),
    )(partials, gamma.reshape(1, C_OUT), beta.reshape(1, C_OUT))
    return out


# speedup vs baseline: 1.3794x; 1.0151x over previous
import functools

import jax
import jax.numpy as jnp
from jax import lax
from jax.experimental import pallas as pl
from jax.experimental.pallas import tpu as pltpu
from jax.experimental.pallas import tpu_sc as plsc

N = 10000
E = 320000
C_IN = 128
C_OUT = 128
KV = 27

NC, NS = 2, 16
NW = NC * NS
BE = 128
CH0, CH1 = 128, 32
TCH = NS * (CH0 + CH1)
PH = 32
EPAD = TCH * BE
NPAD = N + 112
RPT = NPAD // NS


def _mm_body(x_ref, w_ref, o_ref):
    for k in range(KV):
        o_ref[k] = jnp.dot(x_ref[...], w_ref[k],
                           preferred_element_type=jnp.float32)


def _bn_body(p_ref, g_ref, b_ref, o_ref):
    s = p_ref[0, :N, :] + p_ref[1, :N, :]
    mean = jnp.mean(s, axis=0, keepdims=True)
    var = jnp.mean(s * s, axis=0, keepdims=True) - mean * mean
    inv = lax.rsqrt(var + 1e-5)
    o_ref[...] = jnp.maximum((s - mean) * inv * g_ref[...] + b_ref[...], 0.0)


def _sc_scatter_body(xw_hbm, g_hbm, d_hbm, out_hbm,
                     g_v, d_v, rows0_v, rows1_v, acc_sh, sem0, sem1):
    c = lax.axis_index("c")
    s = lax.axis_index("s")

    rows = (rows0_v, rows1_v)
    sems = (sem0, sem1)

    def zbody(i, carry):
        rows0_v[i >> 3, pl.ds((i & 7) * 16, 16)] = jnp.zeros((16,),
                                                             jnp.float32)
        return carry

    lax.fori_loop(0, BE * 8, zbody, 0)
    for t in range(4):
        pltpu.sync_copy(rows0_v,
                        acc_sh.at[pl.ds(s * RPT + t * BE, BE)])
    pltpu.sync_copy(rows0_v.at[pl.ds(0, RPT - 4 * BE)],
                    acc_sh.at[pl.ds(s * RPT + 4 * BE, RPT - 4 * BE)])
    plsc.subcore_barrier()

    def _phase(row0):
        pltpu.sync_copy(g_hbm.at[pl.ds(row0, PH)], g_v)
        pltpu.sync_copy(d_hbm.at[pl.ds(row0, PH)], d_v)

        pltpu.async_copy(xw_hbm.at[g_v.at[0]], rows[0], sems[0])

        def body(j, carry):
            for b in range(2):
                i = 2 * j + b
                pltpu.make_async_copy(xw_hbm.at[g_v.at[i]],
                                      rows[b], sems[b]).wait()

                @pl.when(i + 1 < PH)
                def _():
                    pltpu.async_copy(xw_hbm.at[g_v.at[i + 1]],
                                     rows[1 - b], sems[1 - b])

                pltpu.sync_copy(rows[b], acc_sh.at[d_v.at[i]], add=True)
            return carry

        lax.fori_loop(0, PH // 2, body, 0)

    @pl.when(c == 0)
    def _():
        for h in range(CH0 // PH):
            _phase(s * CH0 + h * PH)

    @pl.when(c == 1)
    def _():
        for h in range(CH1 // PH):
            _phase(NS * CH0 + s * CH1 + h * PH)

    plsc.subcore_barrier()
    pltpu.sync_copy(acc_sh.at[pl.ds(s * RPT, RPT)],
                    out_hbm.at[c, pl.ds(s * RPT, RPT)])


def kernel(x, edge_index, kernel_id, W, gamma, beta):
    src = edge_index[0].astype(jnp.int32)
    dst = edge_index[1].astype(jnp.int32)
    kid = kernel_id.astype(jnp.int32)

    n_blk = 10
    bn = N // n_blk
    xw = pl.pallas_call(
        _mm_body,
        grid=(n_blk,),
        in_specs=[pl.BlockSpec((bn, C_IN), lambda i: (i, 0)),
                  pl.BlockSpec((KV, C_IN, C_OUT), lambda i: (0, 0, 0))],
        out_specs=pl.BlockSpec((KV, bn, C_OUT), lambda i: (0, i, 0)),
        out_shape=jax.ShapeDtypeStruct((KV, N, C_OUT), jnp.float32),
    )(x, W)
    xw_rows = xw.reshape(KV * N, C_OUT)

    g = kid * N + src
    pad = EPAD - E
    g2 = jnp.concatenate([g, jnp.zeros((pad,), jnp.int32)]).reshape(TCH, BE)
    d2 = jnp.concatenate([dst, jnp.full((pad,), N, jnp.int32)]).reshape(TCH, BE)

    mesh = plsc.VectorSubcoreMesh(core_axis_name="c", subcore_axis_name="s")
    partials = pl.kernel(
        _sc_scatter_body,
        out_type=jax.ShapeDtypeStruct((NC, NPAD, C_OUT), jnp.float32),
        mesh=mesh,
        scratch_types=[
            pltpu.VMEM((PH, BE), jnp.int32),
            pltpu.VMEM((PH, BE), jnp.int32),
            pltpu.VMEM((BE, C_OUT), jnp.float32),
            pltpu.VMEM((BE, C_OUT), jnp.float32),
            pltpu.VMEM_SHARED((NPAD, C_OUT), jnp.float32),
            pltpu.SemaphoreType.DMA,
            pltpu.SemaphoreType.DMA,
        ],
    )(xw_rows, g2, d2)

    out = pl.pallas_call(
        _bn_body,
        in_specs=[pl.BlockSpec((NC, NPAD, C_OUT), lambda: (0, 0, 0)),
                  pl.BlockSpec((1, C_OUT), lambda: (0, 0)),
                  pl.BlockSpec((1, C_OUT), lambda: (0, 0))],
        out_specs=pl.BlockSpec((N, C_OUT), lambda: (0, 0)),
        out_shape=jax.ShapeDtypeStruct((N, C_OUT), jnp.float32),
    )(partials, gamma.reshape(1, C_OUT), beta.reshape(1, C_OUT))
    return out


# speedup vs baseline: 1.3795x; 1.0001x over previous
import jax
import jax.numpy as jnp
from jax import lax
from jax.experimental import pallas as pl
from jax.experimental.pallas import tpu as pltpu
from jax.experimental.pallas import tpu_sc as plsc

N = 10000
E = 320000
C_IN = 128
C_OUT = 128
KV = 27

NC, NS = 2, 16
NW = NC * NS
BE = 128
CH0, CH1 = 128, 32
TCH = NS * (CH0 + CH1)
PH = 32
EPAD = TCH * BE
NPAD = N + 112
RPT = NPAD // NS


def _mm_body(x_ref, w_ref, o_ref):
    for k in range(KV):
        o_ref[k] = jnp.dot(x_ref[...], w_ref[k],
                           preferred_element_type=jnp.float32)


def _bn_body(p_ref, g_ref, b_ref, o_ref):
    s = p_ref[0, :N, :] + p_ref[1, :N, :]
    mean = jnp.mean(s, axis=0, keepdims=True)
    var = jnp.mean(s * s, axis=0, keepdims=True) - mean * mean
    inv = lax.rsqrt(var + 1e-5)
    o_ref[...] = jnp.maximum((s - mean) * inv * g_ref[...] + b_ref[...], 0.0)


def _sc_scatter_body(xw_hbm, g_hbm, d_hbm, out_hbm,
                     g_v, d_v, rows0_v, rows1_v, acc_sh, sem0, sem1):
    c = lax.axis_index("c")
    s = lax.axis_index("s")

    rows = (rows0_v, rows1_v)
    sems = (sem0, sem1)

    def zbody(i, carry):
        rows0_v[i >> 3, pl.ds((i & 7) * 16, 16)] = jnp.zeros((16,),
                                                             jnp.float32)
        return carry

    lax.fori_loop(0, BE * 8, zbody, 0)
    for t in range(4):
        pltpu.sync_copy(rows0_v,
                        acc_sh.at[pl.ds(s * RPT + t * BE, BE)])
    pltpu.sync_copy(rows0_v.at[pl.ds(0, RPT - 4 * BE)],
                    acc_sh.at[pl.ds(s * RPT + 4 * BE, RPT - 4 * BE)])
    plsc.subcore_barrier()

    def _phase(row0):
        pltpu.sync_copy(g_hbm.at[pl.ds(row0, PH)], g_v)
        pltpu.sync_copy(d_hbm.at[pl.ds(row0, PH)], d_v)

        pltpu.async_copy(xw_hbm.at[g_v.at[0]], rows[0], sems[0])

        def body(j, carry):
            for b in range(2):
                i = 2 * j + b
                pltpu.make_async_copy(xw_hbm.at[g_v.at[i]],
                                      rows[b], sems[b]).wait()

                @pl.when(i + 1 < PH)
                def _():
                    pltpu.async_copy(xw_hbm.at[g_v.at[i + 1]],
                                     rows[1 - b], sems[1 - b])

                pltpu.sync_copy(rows[b], acc_sh.at[d_v.at[i]], add=True)
            return carry

        lax.fori_loop(0, PH // 2, body, 0)

    @pl.when(c == 0)
    def _():
        for h in range(CH0 // PH):
            _phase(s * CH0 + h * PH)

    @pl.when(c == 1)
    def _():
        for h in range(CH1 // PH):
            _phase(NS * CH0 + s * CH1 + h * PH)

    plsc.subcore_barrier()
    pltpu.sync_copy(acc_sh.at[pl.ds(s * RPT, RPT)],
                    out_hbm.at[c, pl.ds(s * RPT, RPT)])


def kernel(x, edge_index, kernel_id, W, gamma, beta):
    src = edge_index[0].astype(jnp.int32)
    dst = edge_index[1].astype(jnp.int32)
    kid = kernel_id.astype(jnp.int32)

    n_blk = 10
    bn = N // n_blk
    xw = pl.pallas_call(
        _mm_body,
        grid=(n_blk,),
        in_specs=[pl.BlockSpec((bn, C_IN), lambda i: (i, 0)),
                  pl.BlockSpec((KV, C_IN, C_OUT), lambda i: (0, 0, 0))],
        out_specs=pl.BlockSpec((KV, bn, C_OUT), lambda i: (0, i, 0)),
        out_shape=jax.ShapeDtypeStruct((KV, N, C_OUT), jnp.float32),
    )(x, W)
    xw_rows = xw.reshape(KV * N, C_OUT)

    g = kid * N + src
    pad = EPAD - E
    g2 = jnp.concatenate([g, jnp.zeros((pad,), jnp.int32)]).reshape(TCH, BE)
    d2 = jnp.concatenate([dst, jnp.full((pad,), N, jnp.int32)]).reshape(TCH, BE)

    mesh = plsc.VectorSubcoreMesh(core_axis_name="c", subcore_axis_name="s")
    partials = pl.kernel(
        _sc_scatter_body,
        out_type=jax.ShapeDtypeStruct((NC, NPAD, C_OUT), jnp.float32),
        mesh=mesh,
        scratch_types=[
            pltpu.VMEM((PH, BE), jnp.int32),
            pltpu.VMEM((PH, BE), jnp.int32),
            pltpu.VMEM((BE, C_OUT), jnp.float32),
            pltpu.VMEM((BE, C_OUT), jnp.float32),
            pltpu.VMEM_SHARED((NPAD, C_OUT), jnp.float32),
            pltpu.SemaphoreType.DMA,
            pltpu.SemaphoreType.DMA,
        ],
    )(xw_rows, g2, d2)

    out = pl.pallas_call(
        _bn_body,
        in_specs=[pl.BlockSpec((NC, NPAD, C_OUT), lambda: (0, 0, 0)),
                  pl.BlockSpec((1, C_OUT), lambda: (0, 0)),
                  pl.BlockSpec((1, C_OUT), lambda: (0, 0))],
        out_specs=pl.BlockSpec((N, C_OUT), lambda: (0, 0)),
        out_shape=jax.ShapeDtypeStruct((N, C_OUT), jnp.float32),
    )(partials, gamma.reshape(1, C_OUT), beta.reshape(1, C_OUT))
    return out
